# Initial kernel scaffold; baseline (speedup 1.0000x reference)
import jax
import jax.numpy as jnp
from jax.experimental import pallas as pl


def kernel(input, context, hidden, vocab_logits, attn_dist, oov_ids, extra_zeros, W_h, W_c, W_x, b_x):
    raise NotImplementedError("write your pallas kernel here")


# speedup vs baseline: 1.7919x; 1.7919x over previous
import functools

import jax
import jax.numpy as jnp
from jax import lax
from jax.experimental import pallas as pl
from jax.experimental.pallas import tpu as pltpu
from jax.experimental.pallas import tpu_sc as plsc

HIDDEN = 512
CTX = 512
VOCAB = 32000
SRC = 400
NOOV = 64
WIDTH = VOCAB + NOOV
WPAD = 32128
EPS = 1e-30

SRC_PAD = 512
NCHUNK = SRC_PAD // 128

_NC = 2
_NS = 16
_NW = _NC * _NS

ROWS_BLK = 8


def _mix_body(inp_ref, hid_ref, ctx_ref, wl_ref, attn_ref, ez_ref,
              wh_ref, wc_ref, wxb_ref,
              p_ref, attn_out_ref, pgen_ref):
    hp = jnp.sum(hid_ref[...] * wh_ref[...], axis=1, keepdims=True)
    cp = jnp.sum(ctx_ref[...] * wc_ref[...], axis=1, keepdims=True)
    x = inp_ref[...]
    pgen = jax.nn.sigmoid(hp + cp + x * wxb_ref[0, 0] + wxb_ref[0, 1])
    l = wl_ref[...]
    m = jnp.max(l, axis=1, keepdims=True)
    e = jnp.exp(l - m)
    s = jnp.sum(e, axis=1, keepdims=True)
    p_ref[:, :VOCAB] = (pgen / s) * e
    p_ref[:, VOCAB:] = jnp.concatenate(
        [ez_ref[...], jnp.zeros((ROWS_BLK, WPAD - WIDTH), jnp.float32)],
        axis=1)
    attn_out_ref[...] = (1.0 - pgen) * attn_ref[...]
    pgen_ref[...] = pgen


def _log_body(p_ref, y_ref):
    norm = 1.0 + WIDTH * EPS
    y_ref[...] = jnp.log((p_ref[:, :WIDTH] + EPS) / norm)


def _sc_scatter_body(p_hbm, idx_hbm, val_hbm, out_hbm, idx_v, val_v, buf):
    cid = lax.axis_index("c")
    sid = lax.axis_index("s")
    wid = sid * _NC + cid
    n_rows = idx_hbm.shape[0]
    rpw = n_rows // _NW
    base = wid * rpw
    slot = buf.at[pl.ds(sid * WPAD, WPAD)]

    def row_step(k, carry):
        i = base + k
        pltpu.sync_copy(p_hbm.at[pl.ds(i * WPAD, WPAD)], slot)
        pltpu.sync_copy(idx_hbm.at[i], idx_v)
        pltpu.sync_copy(val_hbm.at[i], val_v)
        for jc in range(NCHUNK):
            pltpu.sync_copy(val_v.at[jc], buf.at[idx_v.at[jc]], add=True)
        pltpu.sync_copy(slot, out_hbm.at[pl.ds(i * WPAD, WPAD)])
        return carry

    lax.fori_loop(0, rpw, row_step, 0)


def kernel(input, context, hidden, vocab_logits, attn_dist, oov_ids,
           extra_zeros, W_h, W_c, W_x, b_x):
    n = vocab_logits.shape[0]
    grid = n // ROWS_BLK

    inp_f = input.reshape(n, 1).astype(jnp.float32)
    attn_pad = jnp.pad(attn_dist, ((0, 0), (0, SRC_PAD - SRC)))
    wxb = jnp.concatenate([W_x.reshape(1, 1), b_x.reshape(1, 1)], axis=1)

    row_blk = lambda i: (i, 0)
    fixed = lambda i: (0, 0)
    p, attn_full, pgen = pl.pallas_call(
        _mix_body,
        grid=(grid,),
        in_specs=[
            pl.BlockSpec((ROWS_BLK, 1), row_blk),
            pl.BlockSpec((ROWS_BLK, HIDDEN), row_blk),
            pl.BlockSpec((ROWS_BLK, CTX), row_blk),
            pl.BlockSpec((ROWS_BLK, VOCAB), row_blk),
            pl.BlockSpec((ROWS_BLK, SRC_PAD), row_blk),
            pl.BlockSpec((ROWS_BLK, NOOV), row_blk),
            pl.BlockSpec((1, HIDDEN), fixed),
            pl.BlockSpec((1, CTX), fixed),
            pl.BlockSpec((1, 2), fixed, memory_space=pltpu.SMEM),
        ],
        out_specs=[
            pl.BlockSpec((ROWS_BLK, WPAD), row_blk),
            pl.BlockSpec((ROWS_BLK, SRC_PAD), row_blk),
            pl.BlockSpec((ROWS_BLK, 1), row_blk),
        ],
        out_shape=[
            jax.ShapeDtypeStruct((n, WPAD), jnp.float32),
            jax.ShapeDtypeStruct((n, SRC_PAD), jnp.float32),
            jax.ShapeDtypeStruct((n, 1), jnp.float32),
        ],
    )(inp_f, hidden, context, vocab_logits, attn_pad, extra_zeros,
      W_h, W_c, wxb)

    rpw = n // _NW
    sid_of_row = (jnp.arange(n, dtype=jnp.int32) // rpw) // _NC
    idx = jnp.pad(oov_ids, ((0, 0), (0, SRC_PAD - SRC)))
    idx = (idx + sid_of_row[:, None] * WPAD).reshape(n, NCHUNK, 128)

    mesh = plsc.VectorSubcoreMesh(core_axis_name="c", subcore_axis_name="s",
                                  num_cores=_NC, num_subcores=_NS)
    scatter = pl.kernel(
        _sc_scatter_body,
        out_type=jax.ShapeDtypeStruct((n * WPAD,), jnp.float32),
        mesh=mesh,
        scratch_types=[
            pltpu.VMEM((NCHUNK, 128), jnp.int32),
            pltpu.VMEM((NCHUNK, 128), jnp.float32),
            pltpu.VMEM_SHARED((_NS * WPAD,), jnp.float32),
        ],
    )
    val = attn_full.reshape(n, NCHUNK, 128)
    p2 = scatter(p.reshape(n * WPAD), idx, val).reshape(n, WPAD)

    final = pl.pallas_call(
        _log_body,
        grid=(grid,),
        in_specs=[pl.BlockSpec((ROWS_BLK, WPAD), row_blk)],
        out_specs=pl.BlockSpec((ROWS_BLK, WIDTH), row_blk),
        out_shape=jax.ShapeDtypeStruct((n, WIDTH), jnp.float32),
    )(p2)

    return (final, attn_full[:, :SRC], pgen)


# speedup vs baseline: 3.0550x; 1.7049x over previous
import functools

import jax
import jax.numpy as jnp
from jax import lax
from jax.experimental import pallas as pl
from jax.experimental.pallas import tpu as pltpu
from jax.experimental.pallas import tpu_sc as plsc

HIDDEN = 512
CTX = 512
VOCAB = 32000
SRC = 400
NOOV = 64
WIDTH = VOCAB + NOOV
NT = 251
EPS = 1e-30

SRC_PAD = 512
NCHUNK = SRC_PAD // 128

_NC = 2
_NS = 16
_NW = _NC * _NS

ROWS_BLK = 8


def _sc_delta_body(idx_hbm, val_hbm, delta_hbm, idx_v, val_v, buf):
    cid = lax.axis_index("c")
    sid = lax.axis_index("s")
    wid = sid * _NC + cid
    n_rows = idx_hbm.shape[0]
    rpw = n_rows // _NW
    base = wid * rpw
    zero16 = jnp.zeros((16,), jnp.float32)

    def zinit(t, c):
        for q in range(8):
            buf[t, pl.ds(q * 16, 16)] = zero16
        return c

    lax.fori_loop(0, NT, zinit, 0)

    iota16 = lax.iota(jnp.int32, 16)
    im1 = jnp.maximum(iota16 - 1, 0)
    ip1 = jnp.minimum(iota16 + 1, 15)

    def row_step(k, carry):
        i = base + k
        g = i // ROWS_BLK
        r = i - g * ROWS_BLK
        pltpu.sync_copy(idx_hbm.at[i], idx_v)
        pltpu.sync_copy(val_hbm.at[i], val_v)
        for jc in range(NCHUNK):
            for q in range(8):
                iv = idx_v[jc, pl.ds(q * 16, 16)]
                vv = val_v[jc, pl.ds(q * 16, 16)]
                sk, sv = plsc.sort_key_val(iv, vv)
                nxt = sk.at[ip1].get(mode="promise_in_bounds")
                is_end = (iota16 == 15) | (sk != nxt)
                csum = plsc.cumsum(sv)
                w = jnp.where(is_end, csum, 0.0)
                wsh = jnp.where(iota16 > 0,
                                w.at[im1].get(mode="promise_in_bounds"),
                                0.0)
                tot = csum - plsc.cummax(wsh)
                plsc.addupdate_scatter(
                    buf, [sk >> 7, sk & 127], tot, mask=is_end)
        pltpu.sync_copy(buf, delta_hbm.at[g, :, r])
        for jc in range(NCHUNK):
            for q in range(8):
                iv = idx_v[jc, pl.ds(q * 16, 16)]
                plsc.store_scatter(buf, [iv >> 7, iv & 127], zero16)
        return carry

    lax.fori_loop(0, rpw, row_step, 0)


def _fuse_body(inp_ref, hid_ref, ctx_ref, wl_ref, attn_ref, ez_ref, d_ref,
               wh_ref, wc_ref, wxb_ref,
               y_ref, attn_out_ref, pgen_ref):
    def _bf(v):
        return v.astype(jnp.bfloat16).astype(jnp.float32)

    hp = jnp.sum(_bf(hid_ref[...]) * _bf(wh_ref[...]), axis=1, keepdims=True)
    cp = jnp.sum(_bf(ctx_ref[...]) * _bf(wc_ref[...]), axis=1, keepdims=True)
    x = _bf(inp_ref[...])
    pgen = jax.nn.sigmoid(hp + cp + x * wxb_ref[0, 0] + wxb_ref[0, 1])
    l = wl_ref[...]
    m = jnp.max(l, axis=1, keepdims=True)
    e = jnp.exp(l - m)
    s = jnp.sum(e, axis=1, keepdims=True)
    ps = pgen / s
    sc = 1.0 - pgen
    norm = 1.0 + WIDTH * EPS
    for t in range(NT - 1):
        y_ref[:, 128 * t:128 * (t + 1)] = jnp.log(
            (ps * e[:, 128 * t:128 * (t + 1)] + sc * d_ref[0, t] + EPS)
            / norm)
    y_ref[:, VOCAB:WIDTH] = jnp.log(
        (ez_ref[...] + sc * d_ref[0, NT - 1][:, :NOOV] + EPS) / norm)
    attn_out_ref[...] = sc * attn_ref[...]
    pgen_ref[...] = pgen


def kernel(input, context, hidden, vocab_logits, attn_dist, oov_ids,
           extra_zeros, W_h, W_c, W_x, b_x):
    n = vocab_logits.shape[0]
    grid = n // ROWS_BLK

    inp_f = input.reshape(n, 1).astype(jnp.float32)
    attn_pad = jnp.pad(attn_dist, ((0, 0), (0, SRC_PAD - SRC)))
    wx_bf = W_x.astype(jnp.bfloat16).astype(jnp.float32)
    wxb = jnp.concatenate([wx_bf.reshape(1, 1), b_x.reshape(1, 1)], axis=1)
    idx = jnp.pad(oov_ids, ((0, 0), (0, SRC_PAD - SRC)))
    idx3 = idx.reshape(n, NCHUNK, 128)
    val3 = attn_pad.reshape(n, NCHUNK, 128)

    mesh = plsc.VectorSubcoreMesh(core_axis_name="c", subcore_axis_name="s",
                                  num_cores=_NC, num_subcores=_NS)
    delta = pl.kernel(
        _sc_delta_body,
        out_type=jax.ShapeDtypeStruct((grid, NT, ROWS_BLK, 128), jnp.float32),
        mesh=mesh,
        compiler_params=pltpu.CompilerParams(needs_layout_passes=False),
        scratch_types=[
            pltpu.VMEM((NCHUNK, 128), jnp.int32),
            pltpu.VMEM((NCHUNK, 128), jnp.float32),
            pltpu.VMEM((NT, 128), jnp.float32),
        ],
    )(idx3, val3)

    row_blk = lambda i: (i, 0)
    fixed = lambda i: (0, 0)
    final, attn_full, pgen = pl.pallas_call(
        _fuse_body,
        grid=(grid,),
        in_specs=[
            pl.BlockSpec((ROWS_BLK, 1), row_blk),
            pl.BlockSpec((ROWS_BLK, HIDDEN), row_blk),
            pl.BlockSpec((ROWS_BLK, CTX), row_blk),
            pl.BlockSpec((ROWS_BLK, VOCAB), row_blk),
            pl.BlockSpec((ROWS_BLK, SRC_PAD), row_blk),
            pl.BlockSpec((ROWS_BLK, NOOV), row_blk),
            pl.BlockSpec((1, NT, ROWS_BLK, 128), lambda i: (i, 0, 0, 0)),
            pl.BlockSpec((1, HIDDEN), fixed),
            pl.BlockSpec((1, CTX), fixed),
            pl.BlockSpec((1, 2), fixed, memory_space=pltpu.SMEM),
        ],
        out_specs=[
            pl.BlockSpec((ROWS_BLK, WIDTH), row_blk),
            pl.BlockSpec((ROWS_BLK, SRC_PAD), row_blk),
            pl.BlockSpec((ROWS_BLK, 1), row_blk),
        ],
        out_shape=[
            jax.ShapeDtypeStruct((n, WIDTH), jnp.float32),
            jax.ShapeDtypeStruct((n, SRC_PAD), jnp.float32),
            jax.ShapeDtypeStruct((n, 1), jnp.float32),
        ],
    )(inp_f, hidden, context, vocab_logits, attn_pad, extra_zeros, delta,
      W_h, W_c, wxb)

    return (final, attn_full[:, :SRC], pgen)


# speedup vs baseline: 3.4822x; 1.1399x over previous
import functools

import jax
import jax.numpy as jnp
from jax import lax
from jax.experimental import pallas as pl
from jax.experimental.pallas import tpu as pltpu
from jax.experimental.pallas import tpu_sc as plsc

HIDDEN = 512
CTX = 512
VOCAB = 32000
SRC = 400
NOOV = 64
WIDTH = VOCAB + NOOV
NT = 251
EPS = 1e-30

SRC_PAD = 512
NCHUNK = SRC_PAD // 128

_NC = 2
_NS = 16
_NW = _NC * _NS

ROWS_BLK = 16


def _sc_delta_body(idx_hbm, val_hbm, delta_hbm, idx_v, val_v, buf):
    cid = lax.axis_index("c")
    sid = lax.axis_index("s")
    wid = sid * _NC + cid
    n_rows = idx_hbm.shape[0]
    rpw = n_rows // _NW
    base = wid * rpw
    zero16 = jnp.zeros((16,), jnp.float32)

    def zinit(t, c):
        for q in range(8):
            buf[t, pl.ds(q * 16, 16)] = zero16
        return c

    lax.fori_loop(0, NT, zinit, 0)

    iota16 = lax.iota(jnp.int32, 16)
    im1 = jnp.maximum(iota16 - 1, 0)
    ip1 = jnp.minimum(iota16 + 1, 15)

    def row_step(k, carry):
        i = base + k
        g = i // ROWS_BLK
        r = i - g * ROWS_BLK
        pltpu.sync_copy(idx_hbm.at[i], idx_v)
        pltpu.sync_copy(val_hbm.at[i], val_v)
        for jc in range(NCHUNK):
            for q in range(8):
                iv = idx_v[jc, pl.ds(q * 16, 16)]
                vv = val_v[jc, pl.ds(q * 16, 16)]
                sk, sv = plsc.sort_key_val(iv, vv)
                nxt = sk.at[ip1].get(mode="promise_in_bounds")
                is_end = (iota16 == 15) | (sk != nxt)
                csum = plsc.cumsum(sv)
                w = jnp.where(is_end, csum, 0.0)
                wsh = jnp.where(iota16 > 0,
                                w.at[im1].get(mode="promise_in_bounds"),
                                0.0)
                tot = csum - plsc.cummax(wsh)
                plsc.addupdate_scatter(
                    buf, [sk >> 7, sk & 127], tot, mask=is_end)
        pltpu.sync_copy(buf, delta_hbm.at[g, :, r])
        for jc in range(NCHUNK):
            for q in range(8):
                iv = idx_v[jc, pl.ds(q * 16, 16)]
                plsc.store_scatter(buf, [iv >> 7, iv & 127], zero16)
        return carry

    lax.fori_loop(0, rpw, row_step, 0)


def _fuse_body(inp_ref, hid_ref, ctx_ref, wl_ref, attn_ref, ez_ref, d_ref,
               wh_ref, wc_ref, wxb_ref,
               y_ref, attn_out_ref, pgen_ref):
    def _bf(v):
        return v.astype(jnp.bfloat16).astype(jnp.float32)

    hp = jnp.sum(_bf(hid_ref[...]) * _bf(wh_ref[...]), axis=1, keepdims=True)
    cp = jnp.sum(_bf(ctx_ref[...]) * _bf(wc_ref[...]), axis=1, keepdims=True)
    x = _bf(inp_ref[...])
    pgen = jax.nn.sigmoid(hp + cp + x * wxb_ref[0, 0] + wxb_ref[0, 1])
    l = wl_ref[...]
    m = jnp.max(l, axis=1, keepdims=True)
    e = jnp.exp(l - m)
    s = jnp.sum(e, axis=1, keepdims=True)
    ps = pgen / s
    sc = 1.0 - pgen
    norm = 1.0 + WIDTH * EPS
    for t in range(NT - 1):
        y_ref[:, 128 * t:128 * (t + 1)] = jnp.log(
            (ps * e[:, 128 * t:128 * (t + 1)] + sc * d_ref[0, t] + EPS)
            / norm)
    y_ref[:, VOCAB:WIDTH] = jnp.log(
        (ez_ref[...] + sc * d_ref[0, NT - 1][:, :NOOV] + EPS) / norm)
    attn_out_ref[...] = sc * attn_ref[...]
    pgen_ref[...] = pgen


def kernel(input, context, hidden, vocab_logits, attn_dist, oov_ids,
           extra_zeros, W_h, W_c, W_x, b_x):
    n = vocab_logits.shape[0]
    grid = n // ROWS_BLK

    inp_f = input.reshape(n, 1).astype(jnp.float32)
    attn_pad = jnp.pad(attn_dist, ((0, 0), (0, SRC_PAD - SRC)))
    wx_bf = W_x.astype(jnp.bfloat16).astype(jnp.float32)
    wxb = jnp.concatenate([wx_bf.reshape(1, 1), b_x.reshape(1, 1)], axis=1)
    idx = jnp.pad(oov_ids, ((0, 0), (0, SRC_PAD - SRC)))
    idx3 = idx.reshape(n, NCHUNK, 128)
    val3 = attn_pad.reshape(n, NCHUNK, 128)

    mesh = plsc.VectorSubcoreMesh(core_axis_name="c", subcore_axis_name="s",
                                  num_cores=_NC, num_subcores=_NS)
    delta = pl.kernel(
        _sc_delta_body,
        out_type=jax.ShapeDtypeStruct((grid, NT, ROWS_BLK, 128), jnp.float32),
        mesh=mesh,
        compiler_params=pltpu.CompilerParams(needs_layout_passes=False),
        scratch_types=[
            pltpu.VMEM((NCHUNK, 128), jnp.int32),
            pltpu.VMEM((NCHUNK, 128), jnp.float32),
            pltpu.VMEM((NT, 128), jnp.float32),
        ],
    )(idx3, val3)

    row_blk = lambda i: (i, 0)
    fixed = lambda i: (0, 0)
    final, attn_full, pgen = pl.pallas_call(
        _fuse_body,
        grid=(grid,),
        in_specs=[
            pl.BlockSpec((ROWS_BLK, 1), row_blk),
            pl.BlockSpec((ROWS_BLK, HIDDEN), row_blk),
            pl.BlockSpec((ROWS_BLK, CTX), row_blk),
            pl.BlockSpec((ROWS_BLK, VOCAB), row_blk),
            pl.BlockSpec((ROWS_BLK, SRC_PAD), row_blk),
            pl.BlockSpec((ROWS_BLK, NOOV), row_blk),
            pl.BlockSpec((1, NT, ROWS_BLK, 128), lambda i: (i, 0, 0, 0)),
            pl.BlockSpec((1, HIDDEN), fixed),
            pl.BlockSpec((1, CTX), fixed),
            pl.BlockSpec((1, 2), fixed, memory_space=pltpu.SMEM),
        ],
        out_specs=[
            pl.BlockSpec((ROWS_BLK, WIDTH), row_blk),
            pl.BlockSpec((ROWS_BLK, SRC_PAD), row_blk),
            pl.BlockSpec((ROWS_BLK, 1), row_blk),
        ],
        out_shape=[
            jax.ShapeDtypeStruct((n, WIDTH), jnp.float32),
            jax.ShapeDtypeStruct((n, SRC_PAD), jnp.float32),
            jax.ShapeDtypeStruct((n, 1), jnp.float32),
        ],
    )(inp_f, hidden, context, vocab_logits, attn_pad, extra_zeros, delta,
      W_h, W_c, wxb)

    return (final, attn_full[:, :SRC], pgen)


# speedup vs baseline: 3.6196x; 1.0395x over previous
import functools

import jax
import jax.numpy as jnp
from jax import lax
from jax.experimental import pallas as pl
from jax.experimental.pallas import tpu as pltpu
from jax.experimental.pallas import tpu_sc as plsc

HIDDEN = 512
CTX = 512
VOCAB = 32000
SRC = 400
NOOV = 64
WIDTH = VOCAB + NOOV
NT = 251
EPS = 1e-30

SRC_PAD = 512
NCHUNK = SRC_PAD // 128

_NC = 2
_NS = 16
_NW = _NC * _NS

ROWS_BLK = 32


def _sc_delta_body(idx_hbm, val_hbm, delta_hbm, idx_v, val_v, buf):
    cid = lax.axis_index("c")
    sid = lax.axis_index("s")
    wid = sid * _NC + cid
    n_rows = idx_hbm.shape[0]
    rpw = n_rows // _NW
    base = wid * rpw
    zero16 = jnp.zeros((16,), jnp.float32)

    def zinit(t, c):
        for q in range(8):
            buf[t, pl.ds(q * 16, 16)] = zero16
        return c

    lax.fori_loop(0, NT, zinit, 0)

    iota16 = lax.iota(jnp.int32, 16)
    im1 = jnp.maximum(iota16 - 1, 0)
    ip1 = jnp.minimum(iota16 + 1, 15)

    def row_step(k, carry):
        i = base + k
        g = i // ROWS_BLK
        r = i - g * ROWS_BLK
        pltpu.sync_copy(idx_hbm.at[i], idx_v)
        pltpu.sync_copy(val_hbm.at[i], val_v)
        for jc in range(NCHUNK):
            for q in range(8):
                iv = idx_v[jc, pl.ds(q * 16, 16)]
                vv = val_v[jc, pl.ds(q * 16, 16)]
                sk, sv = plsc.sort_key_val(iv, vv)
                nxt = sk.at[ip1].get(mode="promise_in_bounds")
                is_end = (iota16 == 15) | (sk != nxt)
                csum = plsc.cumsum(sv)
                w = jnp.where(is_end, csum, 0.0)
                wsh = jnp.where(iota16 > 0,
                                w.at[im1].get(mode="promise_in_bounds"),
                                0.0)
                tot = csum - plsc.cummax(wsh)
                plsc.addupdate_scatter(
                    buf, [sk >> 7, sk & 127], tot, mask=is_end)
        pltpu.sync_copy(buf, delta_hbm.at[g, :, r])
        for jc in range(NCHUNK):
            for q in range(8):
                iv = idx_v[jc, pl.ds(q * 16, 16)]
                plsc.store_scatter(buf, [iv >> 7, iv & 127], zero16)
        return carry

    lax.fori_loop(0, rpw, row_step, 0)


def _fuse_body(inp_ref, hid_ref, ctx_ref, wl_ref, attn_ref, ez_ref, d_ref,
               wh_ref, wc_ref, wxb_ref,
               y_ref, attn_out_ref, pgen_ref):
    def _bf(v):
        return v.astype(jnp.bfloat16).astype(jnp.float32)

    hp = jnp.sum(_bf(hid_ref[...]) * _bf(wh_ref[...]), axis=1, keepdims=True)
    cp = jnp.sum(_bf(ctx_ref[...]) * _bf(wc_ref[...]), axis=1, keepdims=True)
    x = _bf(inp_ref[...])
    pgen = jax.nn.sigmoid(hp + cp + x * wxb_ref[0, 0] + wxb_ref[0, 1])
    l = wl_ref[...]
    m = jnp.max(l, axis=1, keepdims=True)
    e = jnp.exp(l - m)
    s = jnp.sum(e, axis=1, keepdims=True)
    ps = pgen / s
    sc = 1.0 - pgen
    norm = 1.0 + WIDTH * EPS
    for t in range(NT - 1):
        y_ref[:, 128 * t:128 * (t + 1)] = jnp.log(
            (ps * e[:, 128 * t:128 * (t + 1)] + sc * d_ref[0, t] + EPS)
            / norm)
    y_ref[:, VOCAB:WIDTH] = jnp.log(
        (ez_ref[...] + sc * d_ref[0, NT - 1][:, :NOOV] + EPS) / norm)
    attn_out_ref[...] = sc * attn_ref[...]
    pgen_ref[...] = pgen


def kernel(input, context, hidden, vocab_logits, attn_dist, oov_ids,
           extra_zeros, W_h, W_c, W_x, b_x):
    n = vocab_logits.shape[0]
    grid = n // ROWS_BLK

    inp_f = input.reshape(n, 1).astype(jnp.float32)
    attn_pad = jnp.pad(attn_dist, ((0, 0), (0, SRC_PAD - SRC)))
    wx_bf = W_x.astype(jnp.bfloat16).astype(jnp.float32)
    wxb = jnp.concatenate([wx_bf.reshape(1, 1), b_x.reshape(1, 1)], axis=1)
    idx = jnp.pad(oov_ids, ((0, 0), (0, SRC_PAD - SRC)))
    idx3 = idx.reshape(n, NCHUNK, 128)
    val3 = attn_pad.reshape(n, NCHUNK, 128)

    mesh = plsc.VectorSubcoreMesh(core_axis_name="c", subcore_axis_name="s",
                                  num_cores=_NC, num_subcores=_NS)
    delta = pl.kernel(
        _sc_delta_body,
        out_type=jax.ShapeDtypeStruct((grid, NT, ROWS_BLK, 128), jnp.float32),
        mesh=mesh,
        compiler_params=pltpu.CompilerParams(needs_layout_passes=False),
        scratch_types=[
            pltpu.VMEM((NCHUNK, 128), jnp.int32),
            pltpu.VMEM((NCHUNK, 128), jnp.float32),
            pltpu.VMEM((NT, 128), jnp.float32),
        ],
    )(idx3, val3)

    row_blk = lambda i: (i, 0)
    fixed = lambda i: (0, 0)
    final, attn_full, pgen = pl.pallas_call(
        _fuse_body,
        grid=(grid,),
        in_specs=[
            pl.BlockSpec((ROWS_BLK, 1), row_blk),
            pl.BlockSpec((ROWS_BLK, HIDDEN), row_blk),
            pl.BlockSpec((ROWS_BLK, CTX), row_blk),
            pl.BlockSpec((ROWS_BLK, VOCAB), row_blk),
            pl.BlockSpec((ROWS_BLK, SRC_PAD), row_blk),
            pl.BlockSpec((ROWS_BLK, NOOV), row_blk),
            pl.BlockSpec((1, NT, ROWS_BLK, 128), lambda i: (i, 0, 0, 0)),
            pl.BlockSpec((1, HIDDEN), fixed),
            pl.BlockSpec((1, CTX), fixed),
            pl.BlockSpec((1, 2), fixed, memory_space=pltpu.SMEM),
        ],
        out_specs=[
            pl.BlockSpec((ROWS_BLK, WIDTH), row_blk),
            pl.BlockSpec((ROWS_BLK, SRC_PAD), row_blk),
            pl.BlockSpec((ROWS_BLK, 1), row_blk),
        ],
        out_shape=[
            jax.ShapeDtypeStruct((n, WIDTH), jnp.float32),
            jax.ShapeDtypeStruct((n, SRC_PAD), jnp.float32),
            jax.ShapeDtypeStruct((n, 1), jnp.float32),
        ],
    )(inp_f, hidden, context, vocab_logits, attn_pad, extra_zeros, delta,
      W_h, W_c, wxb)

    return (final, attn_full[:, :SRC], pgen)


# speedup vs baseline: 3.7921x; 1.0477x over previous
import functools

import jax
import jax.numpy as jnp
from jax import lax
from jax.experimental import pallas as pl
from jax.experimental.pallas import tpu as pltpu
from jax.experimental.pallas import tpu_sc as plsc

HIDDEN = 512
CTX = 512
VOCAB = 32000
SRC = 400
NOOV = 64
WIDTH = VOCAB + NOOV
NT = 251
EPS = 1e-30

SRC_PAD = 512
NCHUNK = SRC_PAD // 128

_NC = 2
_NS = 16
_NW = _NC * _NS

ROWS_BLK = 16
K_PIPE = 2


def _sc_delta_body(idx_hbm, val_hbm, delta_hbm, idx_v, val_v, buf,
                   *, chunk_base, chunk_rows):
    cid = lax.axis_index("c")
    sid = lax.axis_index("s")
    wid = sid * _NC + cid
    rpw = chunk_rows // _NW
    base = chunk_base + wid * rpw
    zero16 = jnp.zeros((16,), jnp.float32)

    def zinit(t, c):
        for q in range(8):
            buf[t, pl.ds(q * 16, 16)] = zero16
        return c

    lax.fori_loop(0, NT, zinit, 0)

    iota16 = lax.iota(jnp.int32, 16)
    im1 = jnp.maximum(iota16 - 1, 0)
    ip1 = jnp.minimum(iota16 + 1, 15)

    def row_step(k, carry):
        i = base + k
        li = i - chunk_base
        g = li // ROWS_BLK
        r = li - g * ROWS_BLK
        pltpu.sync_copy(idx_hbm.at[i], idx_v)
        pltpu.sync_copy(val_hbm.at[i], val_v)
        for jc in range(NCHUNK):
            for q in range(8):
                iv = idx_v[jc, pl.ds(q * 16, 16)]
                vv = val_v[jc, pl.ds(q * 16, 16)]
                sk, sv = plsc.sort_key_val(iv, vv)
                nxt = sk.at[ip1].get(mode="promise_in_bounds")
                is_end = (iota16 == 15) | (sk != nxt)
                csum = plsc.cumsum(sv)
                w = jnp.where(is_end, csum, 0.0)
                wsh = jnp.where(iota16 > 0,
                                w.at[im1].get(mode="promise_in_bounds"),
                                0.0)
                tot = csum - plsc.cummax(wsh)
                plsc.addupdate_scatter(
                    buf, [sk >> 7, sk & 127], tot, mask=is_end)
        pltpu.sync_copy(buf, delta_hbm.at[g, :, r])
        for jc in range(NCHUNK):
            for q in range(8):
                iv = idx_v[jc, pl.ds(q * 16, 16)]
                plsc.store_scatter(buf, [iv >> 7, iv & 127], zero16)
        return carry

    lax.fori_loop(0, rpw, row_step, 0)


def _fuse_math(inp_ref, hid_ref, ctx_ref, wl_ref, attn_ref, ez_ref, d_ref,
               wh_ref, wc_ref, wxb_ref,
               y_ref, attn_out_ref, pgen_ref):
    def _bf(v):
        return v.astype(jnp.bfloat16).astype(jnp.float32)

    hp = jnp.sum(_bf(hid_ref[...]) * _bf(wh_ref[...]), axis=1, keepdims=True)
    cp = jnp.sum(_bf(ctx_ref[...]) * _bf(wc_ref[...]), axis=1, keepdims=True)
    x = _bf(inp_ref[...])
    pgen = jax.nn.sigmoid(hp + cp + x * wxb_ref[0, 0] + wxb_ref[0, 1])
    l = wl_ref[...]
    m = jnp.max(l, axis=1, keepdims=True)
    e = jnp.exp(l - m)
    s = jnp.sum(e, axis=1, keepdims=True)
    ps = pgen / s
    sc = 1.0 - pgen
    norm = 1.0 + WIDTH * EPS
    for t in range(NT - 1):
        y_ref[:, 128 * t:128 * (t + 1)] = jnp.log(
            (ps * e[:, 128 * t:128 * (t + 1)] + sc * d_ref[0, t] + EPS)
            / norm)
    y_ref[:, VOCAB:WIDTH] = jnp.log(
        (ez_ref[...] + sc * d_ref[0, NT - 1][:, :NOOV] + EPS) / norm)
    attn_out_ref[...] = sc * attn_ref[...]
    pgen_ref[...] = pgen


def _fuse_body0(*refs):
    _fuse_math(*refs)


def _fuse_body_acc(y_prev_ref, *refs):
    del y_prev_ref
    _fuse_math(*refs)


def kernel(input, context, hidden, vocab_logits, attn_dist, oov_ids,
           extra_zeros, W_h, W_c, W_x, b_x):
    n = vocab_logits.shape[0]
    nck = n // K_PIPE
    gck = nck // ROWS_BLK

    inp_f = input.reshape(n, 1).astype(jnp.float32)
    attn_pad = jnp.pad(attn_dist, ((0, 0), (0, SRC_PAD - SRC)))
    wx_bf = W_x.astype(jnp.bfloat16).astype(jnp.float32)
    wxb = jnp.concatenate([wx_bf.reshape(1, 1), b_x.reshape(1, 1)], axis=1)
    idx = jnp.pad(oov_ids, ((0, 0), (0, SRC_PAD - SRC)))
    idx3 = idx.reshape(n, NCHUNK, 128)
    val3 = attn_pad.reshape(n, NCHUNK, 128)

    mesh = plsc.VectorSubcoreMesh(core_axis_name="c", subcore_axis_name="s",
                                  num_cores=_NC, num_subcores=_NS)
    deltas = []
    for k in range(K_PIPE):
        delta_k = pl.kernel(
            functools.partial(_sc_delta_body,
                              chunk_base=k * nck, chunk_rows=nck),
            out_type=jax.ShapeDtypeStruct((gck, NT, ROWS_BLK, 128),
                                          jnp.float32),
            mesh=mesh,
            compiler_params=pltpu.CompilerParams(needs_layout_passes=False),
            scratch_types=[
                pltpu.VMEM((NCHUNK, 128), jnp.int32),
                pltpu.VMEM((NCHUNK, 128), jnp.float32),
                pltpu.VMEM((NT, 128), jnp.float32),
            ],
        )(idx3, val3)
        deltas.append(delta_k)

    fixed = lambda i: (0, 0)
    y = None
    attn_chunks = []
    pgen_chunks = []
    for k in range(K_PIPE):
        off = k * gck
        rb = lambda i, off=off: (i + off, 0)
        row_local = lambda i: (i, 0)
        in_specs = [
            pl.BlockSpec((ROWS_BLK, 1), rb),
            pl.BlockSpec((ROWS_BLK, HIDDEN), rb),
            pl.BlockSpec((ROWS_BLK, CTX), rb),
            pl.BlockSpec((ROWS_BLK, VOCAB), rb),
            pl.BlockSpec((ROWS_BLK, SRC_PAD), rb),
            pl.BlockSpec((ROWS_BLK, NOOV), rb),
            pl.BlockSpec((1, NT, ROWS_BLK, 128), lambda i: (i, 0, 0, 0)),
            pl.BlockSpec((1, HIDDEN), fixed),
            pl.BlockSpec((1, CTX), fixed),
            pl.BlockSpec((1, 2), fixed, memory_space=pltpu.SMEM),
        ]
        ins = [inp_f, hidden, context, vocab_logits, attn_pad, extra_zeros,
               deltas[k], W_h, W_c, wxb]
        if k == 0:
            body = _fuse_body0
            aliases = {}
        else:
            body = _fuse_body_acc
            in_specs = [pl.BlockSpec(memory_space=pl.ANY)] + in_specs
            ins = [y] + ins
            aliases = {0: 0}
        y, attn_k, pgen_k = pl.pallas_call(
            body,
            grid=(gck,),
            in_specs=in_specs,
            out_specs=[
                pl.BlockSpec((ROWS_BLK, WIDTH), rb),
                pl.BlockSpec((ROWS_BLK, SRC_PAD), row_local),
                pl.BlockSpec((ROWS_BLK, 1), row_local),
            ],
            out_shape=[
                jax.ShapeDtypeStruct((n, WIDTH), jnp.float32),
                jax.ShapeDtypeStruct((nck, SRC_PAD), jnp.float32),
                jax.ShapeDtypeStruct((nck, 1), jnp.float32),
            ],
            input_output_aliases=aliases,
        )(*ins)
        attn_chunks.append(attn_k[:, :SRC])
        pgen_chunks.append(pgen_k)

    return (y, jnp.concatenate(attn_chunks, axis=0),
            jnp.concatenate(pgen_chunks, axis=0))


# speedup vs baseline: 3.8515x; 1.0157x over previous
import functools

import jax
import jax.numpy as jnp
from jax import lax
from jax.experimental import pallas as pl
from jax.experimental.pallas import tpu as pltpu
from jax.experimental.pallas import tpu_sc as plsc

HIDDEN = 512
CTX = 512
VOCAB = 32000
SRC = 400
NOOV = 64
WIDTH = VOCAB + NOOV
NT = 251
EPS = 1e-30

SRC_PAD = 512
NCHUNK = SRC_PAD // 128

_NC = 2
_NS = 16
_NW = _NC * _NS

ROWS_BLK = 16
K_PIPE = 4


def _sc_delta_body(idx_hbm, val_hbm, delta_hbm, idx_v, val_v, buf,
                   *, chunk_base, chunk_rows):
    cid = lax.axis_index("c")
    sid = lax.axis_index("s")
    wid = sid * _NC + cid
    rpw = chunk_rows // _NW
    base = chunk_base + wid * rpw
    zero16 = jnp.zeros((16,), jnp.float32)

    def zinit(t, c):
        for q in range(8):
            buf[t, pl.ds(q * 16, 16)] = zero16
        return c

    lax.fori_loop(0, NT, zinit, 0)

    iota16 = lax.iota(jnp.int32, 16)
    im1 = jnp.maximum(iota16 - 1, 0)
    ip1 = jnp.minimum(iota16 + 1, 15)

    def row_step(k, carry):
        i = base + k
        li = i - chunk_base
        g = li // ROWS_BLK
        r = li - g * ROWS_BLK
        pltpu.sync_copy(idx_hbm.at[i], idx_v)
        pltpu.sync_copy(val_hbm.at[i], val_v)
        for jc in range(NCHUNK):
            for q in range(8):
                iv = idx_v[jc, pl.ds(q * 16, 16)]
                vv = val_v[jc, pl.ds(q * 16, 16)]
                sk, sv = plsc.sort_key_val(iv, vv)
                nxt = sk.at[ip1].get(mode="promise_in_bounds")
                is_end = (iota16 == 15) | (sk != nxt)
                csum = plsc.cumsum(sv)
                w = jnp.where(is_end, csum, 0.0)
                wsh = jnp.where(iota16 > 0,
                                w.at[im1].get(mode="promise_in_bounds"),
                                0.0)
                tot = csum - plsc.cummax(wsh)
                plsc.addupdate_scatter(
                    buf, [sk >> 7, sk & 127], tot, mask=is_end)
        pltpu.sync_copy(buf, delta_hbm.at[g, :, r])
        for jc in range(NCHUNK):
            for q in range(8):
                iv = idx_v[jc, pl.ds(q * 16, 16)]
                plsc.store_scatter(buf, [iv >> 7, iv & 127], zero16)
        return carry

    lax.fori_loop(0, rpw, row_step, 0)


def _fuse_math(inp_ref, hid_ref, ctx_ref, wl_ref, attn_ref, ez_ref, d_ref,
               wh_ref, wc_ref, wxb_ref,
               y_ref, attn_out_ref, pgen_ref):
    def _bf(v):
        return v.astype(jnp.bfloat16).astype(jnp.float32)

    hp = jnp.sum(_bf(hid_ref[...]) * _bf(wh_ref[...]), axis=1, keepdims=True)
    cp = jnp.sum(_bf(ctx_ref[...]) * _bf(wc_ref[...]), axis=1, keepdims=True)
    x = _bf(inp_ref[...])
    pgen = jax.nn.sigmoid(hp + cp + x * wxb_ref[0, 0] + wxb_ref[0, 1])
    l = wl_ref[...]
    m = jnp.max(l, axis=1, keepdims=True)
    e = jnp.exp(l - m)
    s = jnp.sum(e, axis=1, keepdims=True)
    ps = pgen / s
    sc = 1.0 - pgen
    norm = 1.0 + WIDTH * EPS
    for t in range(NT - 1):
        y_ref[:, 128 * t:128 * (t + 1)] = jnp.log(
            (ps * e[:, 128 * t:128 * (t + 1)] + sc * d_ref[0, t] + EPS)
            / norm)
    y_ref[:, VOCAB:WIDTH] = jnp.log(
        (ez_ref[...] + sc * d_ref[0, NT - 1][:, :NOOV] + EPS) / norm)
    attn_out_ref[...] = sc * attn_ref[...]
    pgen_ref[...] = pgen


def _fuse_body0(*refs):
    _fuse_math(*refs)


def _fuse_body_acc(y_prev_ref, *refs):
    del y_prev_ref
    _fuse_math(*refs)


def kernel(input, context, hidden, vocab_logits, attn_dist, oov_ids,
           extra_zeros, W_h, W_c, W_x, b_x):
    n = vocab_logits.shape[0]
    nck = n // K_PIPE
    gck = nck // ROWS_BLK

    inp_f = input.reshape(n, 1).astype(jnp.float32)
    attn_pad = jnp.pad(attn_dist, ((0, 0), (0, SRC_PAD - SRC)))
    wx_bf = W_x.astype(jnp.bfloat16).astype(jnp.float32)
    wxb = jnp.concatenate([wx_bf.reshape(1, 1), b_x.reshape(1, 1)], axis=1)
    idx = jnp.pad(oov_ids, ((0, 0), (0, SRC_PAD - SRC)))
    idx3 = idx.reshape(n, NCHUNK, 128)
    val3 = attn_pad.reshape(n, NCHUNK, 128)

    mesh = plsc.VectorSubcoreMesh(core_axis_name="c", subcore_axis_name="s",
                                  num_cores=_NC, num_subcores=_NS)
    deltas = []
    for k in range(K_PIPE):
        delta_k = pl.kernel(
            functools.partial(_sc_delta_body,
                              chunk_base=k * nck, chunk_rows=nck),
            out_type=jax.ShapeDtypeStruct((gck, NT, ROWS_BLK, 128),
                                          jnp.float32),
            mesh=mesh,
            compiler_params=pltpu.CompilerParams(needs_layout_passes=False),
            scratch_types=[
                pltpu.VMEM((NCHUNK, 128), jnp.int32),
                pltpu.VMEM((NCHUNK, 128), jnp.float32),
                pltpu.VMEM((NT, 128), jnp.float32),
            ],
        )(idx3, val3)
        deltas.append(delta_k)

    fixed = lambda i: (0, 0)
    y = None
    attn_chunks = []
    pgen_chunks = []
    for k in range(K_PIPE):
        off = k * gck
        rb = lambda i, off=off: (i + off, 0)
        row_local = lambda i: (i, 0)
        in_specs = [
            pl.BlockSpec((ROWS_BLK, 1), rb),
            pl.BlockSpec((ROWS_BLK, HIDDEN), rb),
            pl.BlockSpec((ROWS_BLK, CTX), rb),
            pl.BlockSpec((ROWS_BLK, VOCAB), rb),
            pl.BlockSpec((ROWS_BLK, SRC_PAD), rb),
            pl.BlockSpec((ROWS_BLK, NOOV), rb),
            pl.BlockSpec((1, NT, ROWS_BLK, 128), lambda i: (i, 0, 0, 0)),
            pl.BlockSpec((1, HIDDEN), fixed),
            pl.BlockSpec((1, CTX), fixed),
            pl.BlockSpec((1, 2), fixed, memory_space=pltpu.SMEM),
        ]
        ins = [inp_f, hidden, context, vocab_logits, attn_pad, extra_zeros,
               deltas[k], W_h, W_c, wxb]
        if k == 0:
            body = _fuse_body0
            aliases = {}
        else:
            body = _fuse_body_acc
            in_specs = [pl.BlockSpec(memory_space=pl.ANY)] + in_specs
            ins = [y] + ins
            aliases = {0: 0}
        y, attn_k, pgen_k = pl.pallas_call(
            body,
            grid=(gck,),
            in_specs=in_specs,
            out_specs=[
                pl.BlockSpec((ROWS_BLK, WIDTH), rb),
                pl.BlockSpec((ROWS_BLK, SRC_PAD), row_local),
                pl.BlockSpec((ROWS_BLK, 1), row_local),
            ],
            out_shape=[
                jax.ShapeDtypeStruct((n, WIDTH), jnp.float32),
                jax.ShapeDtypeStruct((nck, SRC_PAD), jnp.float32),
                jax.ShapeDtypeStruct((nck, 1), jnp.float32),
            ],
            input_output_aliases=aliases,
        )(*ins)
        attn_chunks.append(attn_k[:, :SRC])
        pgen_chunks.append(pgen_k)

    return (y, jnp.concatenate(attn_chunks, axis=0),
            jnp.concatenate(pgen_chunks, axis=0))
